# baseline (speedup 1.0000x reference)
import functools

import jax
import jax.numpy as jnp
from jax import lax
from jax.experimental import pallas as pl
from jax.experimental.pallas import tpu as pltpu
from jax.experimental.pallas import tpu_sc as plsc

_E = 8
_K = 2
_D = 1024
_F = 4096
_S = 2048
_TILE = 256
_NT = 24
_LBUF = _NT * _TILE
_FB = _F // 2


def _gate_body(x_ref, gw_ref, gb_ref, dst_ref, prob_ref, te_ref):
    x = x_ref[...]
    logits = jnp.dot(x, gw_ref[...], preferred_element_type=jnp.float32)
    logits = logits + gb_ref[...]
    m = jnp.max(logits, axis=1, keepdims=True)
    ex = jnp.exp(logits - m)
    probs = ex / jnp.sum(ex, axis=1, keepdims=True)

    ids = lax.broadcasted_iota(jnp.int32, (_S, _E), 1)
    m1 = jnp.max(probs, axis=1, keepdims=True)
    a1 = jnp.min(jnp.where(probs == m1, ids, _E), axis=1, keepdims=True)
    probs_m = jnp.where(ids == a1, -1.0, probs)
    m2 = jnp.max(probs_m, axis=1, keepdims=True)
    a2 = jnp.min(jnp.where(probs_m == m2, ids, _E), axis=1, keepdims=True)
    s = m1 + m2
    p1 = m1 / s
    p2 = m2 / s

    onehot = (ids == a1).astype(jnp.float32) + (ids == a2).astype(jnp.float32)
    csum = onehot
    sh = 1
    while sh < _S:
        shifted = jnp.concatenate(
            [jnp.zeros((sh, _E), jnp.float32), csum[:-sh, :]], axis=0)
        csum = csum + shifted
        sh *= 2
    counts = csum[_S - 1:_S, :]
    padded = jnp.floor((counts + (_TILE - 1)) * (1.0 / _TILE)) * _TILE
    incl = padded
    sh = 1
    while sh < _E:
        shifted = jnp.concatenate(
            [jnp.zeros((1, sh), jnp.float32), incl[:, :-sh]], axis=1)
        incl = incl + shifted
        sh *= 2
    off = incl - padded

    c1 = jnp.sum(jnp.where(ids == a1, csum, 0.0), axis=1, keepdims=True)
    c2 = jnp.sum(jnp.where(ids == a2, csum, 0.0), axis=1, keepdims=True)
    offb = jnp.broadcast_to(off, (_S, _E))
    o1 = jnp.sum(jnp.where(ids == a1, offb, 0.0), axis=1, keepdims=True)
    o2 = jnp.sum(jnp.where(ids == a2, offb, 0.0), axis=1, keepdims=True)
    dst1 = o1 + c1 - 1.0
    dst2 = o2 + c2 - 1.0
    dst_ref[...] = jnp.concatenate([dst1, dst2], axis=1).astype(jnp.int32)
    prob_ref[...] = jnp.concatenate([p1, p2], axis=1)

    tif = lax.broadcasted_iota(jnp.int32, (128, _E), 0).astype(jnp.float32)
    cmp = (tif * float(_TILE) >= jnp.broadcast_to(incl, (128, _E)))
    te = jnp.sum(cmp.astype(jnp.float32), axis=1, keepdims=True)
    te = jnp.minimum(te, float(_E - 1))
    te_ref[...] = te.astype(jnp.int32)


def _gate(xf, gate_W, gate_b):
    return pl.pallas_call(
        _gate_body,
        out_shape=[
            jax.ShapeDtypeStruct((_S, _K), jnp.int32),
            jax.ShapeDtypeStruct((_S, _K), jnp.float32),
            jax.ShapeDtypeStruct((128, 1), jnp.int32),
        ],
    )(xf, gate_W, gate_b.reshape(1, _E))


_PREC = lax.Precision.DEFAULT


def _ffn_a_body(te_ref, x_ref, w1_ref, b1_ref, w2_ref, rw_ref, out_ref):
    h = jnp.dot(x_ref[...], w1_ref[0], preferred_element_type=jnp.float32,
                precision=_PREC)
    h = jnp.maximum(h + b1_ref[0], 0.0)
    o = jnp.dot(h, w2_ref[0], preferred_element_type=jnp.float32,
                precision=_PREC)
    out_ref[...] = o * rw_ref[...]


def _ffn_b_body(te_ref, x_ref, w1_ref, b1_ref, w2_ref, b2_ref, rw_ref,
                ya_ref, out_ref):
    h = jnp.dot(x_ref[...], w1_ref[0], preferred_element_type=jnp.float32,
                precision=_PREC)
    h = jnp.maximum(h + b1_ref[0], 0.0)
    o = jnp.dot(h, w2_ref[0], preferred_element_type=jnp.float32,
                precision=_PREC)
    out_ref[...] = (o + b2_ref[0]) * rw_ref[...] + ya_ref[...]


def _ffn(te, xs, rw, W1, b1, W2, b2):
    rw2 = rw.reshape(_LBUF, 1)
    row_spec = pl.BlockSpec((_TILE, _D), lambda i, te_r: (i, 0))
    rw_spec = pl.BlockSpec((_TILE, 1), lambda i, te_r: (i, 0))
    w1_spec = pl.BlockSpec((1, _D, _FB), lambda i, te_r: (te_r[i], 0, 0))
    b1_spec = pl.BlockSpec((1, 1, _FB), lambda i, te_r: (te_r[i], 0, 0))
    w2_spec = pl.BlockSpec((1, _FB, _D), lambda i, te_r: (te_r[i], 0, 0))
    b2_spec = pl.BlockSpec((1, 1, _D), lambda i, te_r: (te_r[i], 0, 0))
    out_shape = jax.ShapeDtypeStruct((_LBUF, _D), jnp.float32)

    ya = pl.pallas_call(
        _ffn_a_body,
        grid_spec=pltpu.PrefetchScalarGridSpec(
            num_scalar_prefetch=1,
            grid=(_NT,),
            in_specs=[row_spec, w1_spec, b1_spec, w2_spec, rw_spec],
            out_specs=row_spec,
        ),
        out_shape=out_shape,
    )(te, xs, W1[:, :, :_FB], b1[:, None, :_FB], W2[:, :_FB, :], rw2)

    ys = pl.pallas_call(
        _ffn_b_body,
        grid_spec=pltpu.PrefetchScalarGridSpec(
            num_scalar_prefetch=1,
            grid=(_NT,),
            in_specs=[row_spec, w1_spec, b1_spec, w2_spec, b2_spec, rw_spec,
                      row_spec],
            out_specs=row_spec,
        ),
        out_shape=out_shape,
    )(te, xs, W1[:, :, _FB:], b1[:, None, _FB:], W2[:, _FB:, :],
      b2[:, None, :], rw2, ya)
    return ys


_NC = 2
_NS = 16
_NW = _NC * _NS
_ROWS_PER_W = _LBUF // _NW
_GCH = 48
_NA = _S * _K


def _sc_mesh():
    return plsc.VectorSubcoreMesh(core_axis_name="c", subcore_axis_name="s")


_APT = _NA // _NS
_ZPT = _LBUF // _NS
_XPT = _S // _NW


def _dispatch_body(dst3_hbm, probs_hbm, zeros_hbm, d0_hbm, d1_hbm, x_hbm,
                   xs_hbm, rw_hbm,
                   dstv3, pv, d0c, d1c, xv, w_sp, sem):
    s = lax.axis_index("s")
    c = lax.axis_index("c")
    wid = s * _NC + c

    pltpu.sync_copy(dst3_hbm.at[pl.ds(s * 2, 2)], dstv3)
    pltpu.sync_copy(probs_hbm.at[pl.ds(s * _APT, _APT)], pv)
    z0 = s * _ZPT
    pltpu.sync_copy(zeros_hbm.at[pl.ds(z0, _ZPT)], w_sp.at[pl.ds(z0, _ZPT)])
    plsc.subcore_barrier()
    pltpu.sync_copy(pv.at[pl.ds(0, 128)], w_sp.at[dstv3.at[0]], add=True)
    pltpu.sync_copy(pv.at[pl.ds(128, 128)], w_sp.at[dstv3.at[1]], add=True)

    tb = wid * _XPT
    pltpu.sync_copy(d0_hbm.at[pl.ds(tb, _XPT)], d0c)
    pltpu.sync_copy(d1_hbm.at[pl.ds(tb, _XPT)], d1c)
    pltpu.sync_copy(x_hbm.at[pl.ds(tb, _XPT)], xv)
    cp0 = pltpu.async_copy(xv, xs_hbm.at[d0c], sem)
    cp1 = pltpu.async_copy(xv, xs_hbm.at[d1c], sem)
    cp0.wait()
    cp1.wait()

    plsc.subcore_barrier()

    @pl.when(s == 0)
    def _emit_rw():
        pltpu.sync_copy(w_sp, rw_hbm)


def _dispatch(dstf, probsf, xf, d0, d1):
    zeros = jnp.zeros((_LBUF,), jnp.float32)
    f = pl.kernel(
        _dispatch_body,
        out_type=[
            jax.ShapeDtypeStruct((_LBUF, _D), jnp.float32),
            jax.ShapeDtypeStruct((_LBUF,), jnp.float32),
        ],
        mesh=_sc_mesh(),
        scratch_types=[
            pltpu.VMEM((2, 128), jnp.int32),
            pltpu.VMEM((_APT,), jnp.float32),
            pltpu.VMEM((_XPT,), jnp.int32),
            pltpu.VMEM((_XPT,), jnp.int32),
            pltpu.VMEM((_XPT, _D), jnp.float32),
            pltpu.VMEM_SHARED((_LBUF,), jnp.float32),
            pltpu.SemaphoreType.DMA,
        ],
    )
    xs, rw = f(dstf.reshape(_NA // 128, 128), probsf, zeros, d0, d1, xf)
    return xs, rw


_TOK_PER_W = _S // _NW


def _combine_body(ys_hbm, d0_hbm, d1_hbm, out_hbm, d0v, d1v, acc, tmp, sem):
    s = lax.axis_index("s")
    c = lax.axis_index("c")
    wid = s * _NC + c
    tb = wid * _TOK_PER_W
    pltpu.sync_copy(d0_hbm.at[pl.ds(tb, _TOK_PER_W)], d0v)
    pltpu.sync_copy(d1_hbm.at[pl.ds(tb, _TOK_PER_W)], d1v)

    def chunk(ch, carry):
        i0 = d0v[pl.ds(ch * 16, 16)]
        i1 = d1v[pl.ds(ch * 16, 16)]
        pltpu.async_copy(ys_hbm.at[i0], acc, sem).wait()
        pltpu.async_copy(ys_hbm.at[i1], tmp, sem).wait()

        def addrow(r, carry2):
            for k2 in range(_D // 16):
                sl = pl.ds(k2 * 16, 16)
                acc[r, sl] = acc[r, sl] + tmp[r, sl]
            return carry2

        lax.fori_loop(0, 16, addrow, 0)
        pltpu.sync_copy(acc, out_hbm.at[pl.ds(tb + ch * 16, 16)])
        return carry

    lax.fori_loop(0, _TOK_PER_W // 16, chunk, 0)


def _combine(ys, d0, d1):
    f = pl.kernel(
        _combine_body,
        out_type=jax.ShapeDtypeStruct((_S, _D), jnp.float32),
        mesh=_sc_mesh(),
        scratch_types=[
            pltpu.VMEM((_TOK_PER_W,), jnp.int32),
            pltpu.VMEM((_TOK_PER_W,), jnp.int32),
            pltpu.VMEM((16, _D), jnp.float32),
            pltpu.VMEM((16, _D), jnp.float32),
            pltpu.SemaphoreType.DMA,
        ],
    )
    return f(ys, d0, d1)


def kernel(x, gate_W, gate_b, expert_W1, expert_b1, expert_W2, expert_b2):
    bs, sl, d = x.shape
    xf = x.reshape(_S, _D)
    dst, probs, te128 = _gate(xf, gate_W, gate_b)
    te = te128.reshape(-1)[:_NT]
    dstf = dst.reshape(-1)
    probsf = probs.reshape(-1)
    xs, rw = _dispatch(dstf, probsf, xf, dst[:, 0], dst[:, 1])
    ys = _ffn(te, xs, rw, expert_W1, expert_b1, expert_W2, expert_b2)
    out = _combine(ys, dst[:, 0], dst[:, 1])
    return out.reshape(bs, sl, d)


# speedup vs baseline: 1.3425x; 1.3425x over previous
import functools

import jax
import jax.numpy as jnp
from jax import lax
from jax.experimental import pallas as pl
from jax.experimental.pallas import tpu as pltpu
from jax.experimental.pallas import tpu_sc as plsc

_E = 8
_K = 2
_D = 1024
_F = 4096
_S = 2048
_TILE = 256
_NT = 24
_LBUF = _NT * _TILE
_FB = _F // 2


def _gate_body(x_ref, gw_ref, gb_ref, dst_ref, prob_ref, te_ref):
    x = x_ref[...]
    logits = jnp.dot(x, gw_ref[...], preferred_element_type=jnp.float32)
    logits = logits + gb_ref[...]
    m = jnp.max(logits, axis=1, keepdims=True)
    ex = jnp.exp(logits - m)
    probs = ex / jnp.sum(ex, axis=1, keepdims=True)

    ids = lax.broadcasted_iota(jnp.int32, (_S, _E), 1)
    m1 = jnp.max(probs, axis=1, keepdims=True)
    a1 = jnp.min(jnp.where(probs == m1, ids, _E), axis=1, keepdims=True)
    probs_m = jnp.where(ids == a1, -1.0, probs)
    m2 = jnp.max(probs_m, axis=1, keepdims=True)
    a2 = jnp.min(jnp.where(probs_m == m2, ids, _E), axis=1, keepdims=True)
    s = m1 + m2
    p1 = m1 / s
    p2 = m2 / s

    onehot = (ids == a1).astype(jnp.float32) + (ids == a2).astype(jnp.float32)
    csum = onehot
    sh = 1
    while sh < _S:
        shifted = jnp.concatenate(
            [jnp.zeros((sh, _E), jnp.float32), csum[:-sh, :]], axis=0)
        csum = csum + shifted
        sh *= 2
    counts = csum[_S - 1:_S, :]
    padded = jnp.floor((counts + (_TILE - 1)) * (1.0 / _TILE)) * _TILE
    incl = padded
    sh = 1
    while sh < _E:
        shifted = jnp.concatenate(
            [jnp.zeros((1, sh), jnp.float32), incl[:, :-sh]], axis=1)
        incl = incl + shifted
        sh *= 2
    off = incl - padded

    c1 = jnp.sum(jnp.where(ids == a1, csum, 0.0), axis=1, keepdims=True)
    c2 = jnp.sum(jnp.where(ids == a2, csum, 0.0), axis=1, keepdims=True)
    offb = jnp.broadcast_to(off, (_S, _E))
    o1 = jnp.sum(jnp.where(ids == a1, offb, 0.0), axis=1, keepdims=True)
    o2 = jnp.sum(jnp.where(ids == a2, offb, 0.0), axis=1, keepdims=True)
    dst1 = o1 + c1 - 1.0
    dst2 = o2 + c2 - 1.0
    dst_ref[...] = jnp.concatenate([dst1, dst2], axis=1).astype(jnp.int32)
    prob_ref[...] = jnp.concatenate([p1, p2], axis=1)

    tif = lax.broadcasted_iota(jnp.int32, (128, _E), 0).astype(jnp.float32)
    cmp = (tif * float(_TILE) >= jnp.broadcast_to(incl, (128, _E)))
    te = jnp.sum(cmp.astype(jnp.float32), axis=1, keepdims=True)
    te = jnp.minimum(te, float(_E - 1))
    r1 = lax.broadcasted_iota(jnp.int32, (128, 1), 0)
    used = jnp.broadcast_to(incl[:, _E - 1:] * (1.0 / _TILE), (128, 1))
    te = jnp.where(r1 == _NT, used, te)
    te_ref[...] = te.astype(jnp.int32)


def _gate(xf, gate_W, gate_b):
    return pl.pallas_call(
        _gate_body,
        out_shape=[
            jax.ShapeDtypeStruct((_S, _K), jnp.int32),
            jax.ShapeDtypeStruct((_S, _K), jnp.float32),
            jax.ShapeDtypeStruct((128, 1), jnp.int32),
        ],
    )(xf, gate_W, gate_b.reshape(1, _E))


def _ffn_body(te_ref, x_ref, w1_ref, b1_ref, w2_ref, b2_ref, rw_ref, out_ref):
    @pl.when(pl.program_id(0) < te_ref[_NT])
    def _active():
        h = jnp.dot(x_ref[...], w1_ref[0], preferred_element_type=jnp.float32)
        h = jnp.maximum(h + b1_ref[0], 0.0).astype(jnp.bfloat16)
        o = jnp.dot(h, w2_ref[0], preferred_element_type=jnp.float32)
        out_ref[...] = (o + b2_ref[0]) * rw_ref[...]


def _ffn(te, xs, rw, W1, b1, W2, b2):
    rw2 = rw.reshape(_LBUF, 1)
    x_spec = pl.BlockSpec((_TILE, _D), lambda i, te_r: (i, 0))
    rw_spec = pl.BlockSpec((_TILE, 1), lambda i, te_r: (i, 0))
    w1_spec = pl.BlockSpec((1, _D, _F), lambda i, te_r: (te_r[i], 0, 0))
    b1_spec = pl.BlockSpec((1, 1, _F), lambda i, te_r: (te_r[i], 0, 0))
    w2_spec = pl.BlockSpec((1, _F, _D), lambda i, te_r: (te_r[i], 0, 0))
    b2_spec = pl.BlockSpec((1, 1, _D), lambda i, te_r: (te_r[i], 0, 0))
    out_shape = jax.ShapeDtypeStruct((_LBUF, _D), jnp.float32)

    ys = pl.pallas_call(
        _ffn_body,
        grid_spec=pltpu.PrefetchScalarGridSpec(
            num_scalar_prefetch=1,
            grid=(_NT,),
            in_specs=[x_spec, w1_spec, b1_spec, w2_spec, b2_spec, rw_spec],
            out_specs=x_spec,
        ),
        out_shape=out_shape,
    )(te, xs.astype(jnp.bfloat16), W1.astype(jnp.bfloat16), b1[:, None, :],
      W2.astype(jnp.bfloat16), b2[:, None, :], rw2)
    return ys


_NC = 2
_NS = 16
_NW = _NC * _NS
_ROWS_PER_W = _LBUF // _NW
_GCH = 48
_NA = _S * _K


def _sc_mesh():
    return plsc.VectorSubcoreMesh(core_axis_name="c", subcore_axis_name="s")


_APT = _NA // _NS
_ZPT = _LBUF // _NS
_XPT = _S // _NW


def _dispatch_body(dst3_hbm, probs_hbm, zeros_hbm, d0_hbm, d1_hbm, x_hbm,
                   xs_hbm, rw_hbm,
                   dstv3, pv, d0c, d1c, xv, w_sp, sem):
    s = lax.axis_index("s")
    c = lax.axis_index("c")
    wid = s * _NC + c

    pltpu.sync_copy(dst3_hbm.at[pl.ds(s * 2, 2)], dstv3)
    pltpu.sync_copy(probs_hbm.at[pl.ds(s * _APT, _APT)], pv)
    z0 = s * _ZPT
    pltpu.sync_copy(zeros_hbm.at[pl.ds(z0, _ZPT)], w_sp.at[pl.ds(z0, _ZPT)])
    plsc.subcore_barrier()
    pltpu.sync_copy(pv.at[pl.ds(0, 128)], w_sp.at[dstv3.at[0]], add=True)
    pltpu.sync_copy(pv.at[pl.ds(128, 128)], w_sp.at[dstv3.at[1]], add=True)

    tb = wid * _XPT
    pltpu.sync_copy(d0_hbm.at[pl.ds(tb, _XPT)], d0c)
    pltpu.sync_copy(d1_hbm.at[pl.ds(tb, _XPT)], d1c)
    pltpu.sync_copy(x_hbm.at[pl.ds(tb, _XPT)], xv)
    cp0 = pltpu.async_copy(xv, xs_hbm.at[d0c], sem)
    cp1 = pltpu.async_copy(xv, xs_hbm.at[d1c], sem)
    cp0.wait()
    cp1.wait()

    plsc.subcore_barrier()

    @pl.when(s == 0)
    def _emit_rw():
        pltpu.sync_copy(w_sp, rw_hbm)


def _dispatch(dstf, probsf, xf, d0, d1):
    zeros = jnp.zeros((_LBUF,), jnp.float32)
    f = pl.kernel(
        _dispatch_body,
        out_type=[
            jax.ShapeDtypeStruct((_LBUF, _D), jnp.float32),
            jax.ShapeDtypeStruct((_LBUF,), jnp.float32),
        ],
        mesh=_sc_mesh(),
        scratch_types=[
            pltpu.VMEM((2, 128), jnp.int32),
            pltpu.VMEM((_APT,), jnp.float32),
            pltpu.VMEM((_XPT,), jnp.int32),
            pltpu.VMEM((_XPT,), jnp.int32),
            pltpu.VMEM((_XPT, _D), jnp.float32),
            pltpu.VMEM_SHARED((_LBUF,), jnp.float32),
            pltpu.SemaphoreType.DMA,
        ],
    )
    xs, rw = f(dstf.reshape(_NA // 128, 128), probsf, zeros, d0, d1, xf)
    return xs, rw


_TOK_PER_W = _S // _NW


def _combine_body(ys_hbm, d0_hbm, d1_hbm, out_hbm, d0v, d1v, acc, tmp, sem):
    s = lax.axis_index("s")
    c = lax.axis_index("c")
    wid = s * _NC + c
    tb = wid * _TOK_PER_W
    pltpu.sync_copy(d0_hbm.at[pl.ds(tb, _TOK_PER_W)], d0v)
    pltpu.sync_copy(d1_hbm.at[pl.ds(tb, _TOK_PER_W)], d1v)

    def chunk(ch, carry):
        i0 = d0v[pl.ds(ch * 16, 16)]
        i1 = d1v[pl.ds(ch * 16, 16)]
        pltpu.async_copy(ys_hbm.at[i0], acc, sem).wait()
        pltpu.async_copy(ys_hbm.at[i1], tmp, sem).wait()

        def addrow(r, carry2):
            for k2 in range(_D // 16):
                sl = pl.ds(k2 * 16, 16)
                acc[r, sl] = acc[r, sl] + tmp[r, sl]
            return carry2

        lax.fori_loop(0, 16, addrow, 0)
        pltpu.sync_copy(acc, out_hbm.at[pl.ds(tb + ch * 16, 16)])
        return carry

    lax.fori_loop(0, _TOK_PER_W // 16, chunk, 0)


def _combine(ys, d0, d1):
    f = pl.kernel(
        _combine_body,
        out_type=jax.ShapeDtypeStruct((_S, _D), jnp.float32),
        mesh=_sc_mesh(),
        scratch_types=[
            pltpu.VMEM((_TOK_PER_W,), jnp.int32),
            pltpu.VMEM((_TOK_PER_W,), jnp.int32),
            pltpu.VMEM((16, _D), jnp.float32),
            pltpu.VMEM((16, _D), jnp.float32),
            pltpu.SemaphoreType.DMA,
        ],
    )
    return f(ys, d0, d1)


def kernel(x, gate_W, gate_b, expert_W1, expert_b1, expert_W2, expert_b2):
    bs, sl, d = x.shape
    xf = x.reshape(_S, _D)
    dst, probs, te128 = _gate(xf, gate_W, gate_b)
    te = te128.reshape(-1)[:_NT + 1]
    dstf = dst.reshape(-1)
    probsf = probs.reshape(-1)
    xs, rw = _dispatch(dstf, probsf, xf, dst[:, 0], dst[:, 1])
    ys = _ffn(te, xs, rw, expert_W1, expert_b1, expert_W2, expert_b2)
    out = _combine(ys, dst[:, 0], dst[:, 1])
    return out.reshape(bs, sl, d)
